# baseline (speedup 1.0000x reference)
import functools

import jax
import jax.numpy as jnp
from jax import lax
from jax.experimental import pallas as pl
from jax.experimental.pallas import tpu as pltpu
from jax.experimental.pallas import tpu_sc as plsc

VOCAB = 1000000
DIM = 64
B = 16384
F = 27

NUM_CORES = 2
NUM_SUBCORES = 16
NW = NUM_CORES * NUM_SUBCORES
BPW = B // NW
LANES = 16
CHUNK = 128
NCHUNK = BPW // CHUNK


def _emb_body(xf_hbm, w_hbm, out_hbm, x_v, idx_v, emb_v, sem):
    wid = lax.axis_index("s") * NUM_CORES + lax.axis_index("c")
    base = wid * BPW

    pltpu.sync_copy(xf_hbm.at[pl.ds(base * F, BPW * F)], x_v)

    for i in range(BPW // LANES):
        pos = (lax.iota(jnp.int32, LANES) + (i * LANES)) * F + 1
        vals = plsc.load_gather(x_v, [pos])
        j, off = divmod(i * LANES, CHUNK)
        idx_v[j, pl.ds(off, LANES)] = vals.astype(jnp.int32)

    copies = [
        pltpu.async_copy(w_hbm.at[idx_v.at[j]],
                         emb_v.at[pl.ds(j * CHUNK, CHUNK)], sem)
        for j in range(NCHUNK)
    ]
    for c in copies:
        c.wait()

    pltpu.sync_copy(emb_v, out_hbm.at[pl.ds(base, BPW)])


@jax.jit
def kernel(x, W):
    mesh = plsc.VectorSubcoreMesh(core_axis_name="c", subcore_axis_name="s")
    gather = functools.partial(
        pl.kernel,
        mesh=mesh,
        compiler_params=pltpu.CompilerParams(
            needs_layout_passes=False, use_tc_tiling_on_sc=False),
        out_type=jax.ShapeDtypeStruct((B, DIM), jnp.float32),
        scratch_types=[
            pltpu.VMEM((BPW * F,), jnp.float32),
            pltpu.VMEM((NCHUNK, CHUNK), jnp.int32),
            pltpu.VMEM((BPW, DIM), jnp.float32),
            pltpu.SemaphoreType.DMA,
        ],
    )(_emb_body)
    emb = gather(x.reshape(-1), W)
    return jnp.concatenate([x[:, :1], emb, x[:, 2:]], axis=1)
